# Initial kernel scaffold; baseline (speedup 1.0000x reference)
import functools

import jax
import jax.numpy as jnp
from jax import lax
from jax.experimental import pallas as pl
from jax.experimental.pallas import tpu as pltpu
from jax.experimental.pallas import tpu_sc as plsc

N = 10000
E = 320000
D = 128
H = D // 2
NSUB = 16
EB = 80
EPS_DEN = 1e-6
EPS_BN = 1e-5

E_PER_SUB = E // NSUB
NBLK = E_PER_SUB // EB
N_PER_SUB = N // NSUB
ZROWS = 125


def _node_linear_body(h_ref, wa_ref, ba_ref, wdb0_ref, bdb0_ref, wdb1_ref,
                      bdb1_ref, we0_ref, be0_ref, we1_ref, be1_ref,
                      ah_ref, db0_ref, db1_ref, eh0_ref, eh1_ref):
    hv = h_ref[...]
    ah_ref[...] = jnp.dot(hv, wa_ref[...],
                          preferred_element_type=jnp.float32) + ba_ref[...]
    db0_ref[...] = jnp.dot(hv, wdb0_ref[...],
                           preferred_element_type=jnp.float32) + bdb0_ref[...]
    db1_ref[...] = jnp.dot(hv, wdb1_ref[...],
                           preferred_element_type=jnp.float32) + bdb1_ref[...]
    eh0_ref[...] = jnp.dot(hv, we0_ref[...],
                           preferred_element_type=jnp.float32) + be0_ref[...]
    eh1_ref[...] = jnp.dot(hv, we1_ref[...],
                           preferred_element_type=jnp.float32) + be1_ref[...]


def _node_linears(h, W_A, b_A, W_DB0, b_DB0, W_DB1, b_DB1, W_E0, b_E0,
                  W_E1, b_E1):
    f32 = jnp.float32
    return pl.pallas_call(
        _node_linear_body,
        out_shape=[
            jax.ShapeDtypeStruct((N, D), f32),
            jax.ShapeDtypeStruct((N, D), f32),
            jax.ShapeDtypeStruct((N, D), f32),
            jax.ShapeDtypeStruct((N, H), f32),
            jax.ShapeDtypeStruct((N, H), f32),
        ],
    )(h, W_A, b_A.reshape(1, D), W_DB0, b_DB0.reshape(1, D), W_DB1,
      b_DB1.reshape(1, D), W_E0, b_E0.reshape(1, H), W_E1, b_E1.reshape(1, H))


CE_BLK = 2000


def _ce_body(e_ref, wc0_ref, bc0_ref, wc1_ref, bc1_ref, ce0_ref, ce1_ref):
    ev = e_ref[...]
    ce0_ref[...] = jnp.dot(ev, wc0_ref[...],
                           preferred_element_type=jnp.float32) + bc0_ref[...]
    ce1_ref[...] = jnp.dot(ev, wc1_ref[...],
                           preferred_element_type=jnp.float32) + bc1_ref[...]


def _ce_linears(e, W_C0, b_C0, W_C1, b_C1):
    f32 = jnp.float32
    grid = E // CE_BLK
    return pl.pallas_call(
        _ce_body,
        grid=(grid,),
        in_specs=[
            pl.BlockSpec((CE_BLK, D), lambda i: (i, 0)),
            pl.BlockSpec((D, H), lambda i: (0, 0)),
            pl.BlockSpec((1, H), lambda i: (0, 0)),
            pl.BlockSpec((D, H), lambda i: (0, 0)),
            pl.BlockSpec((1, H), lambda i: (0, 0)),
        ],
        out_specs=[
            pl.BlockSpec((CE_BLK, H), lambda i: (i, 0)),
            pl.BlockSpec((CE_BLK, H), lambda i: (i, 0)),
        ],
        out_shape=[
            jax.ShapeDtypeStruct((E, H), f32),
            jax.ShapeDtypeStruct((E, H), f32),
        ],
    )(e, W_C0, b_C0.reshape(1, H), W_C1, b_C1.reshape(1, H))


def _edge_half(sub, src_hbm, dst_hbm, db_tab, eh_tab, ce_hbm,
               eij_hbm, num_hbm, den_hbm, est_hbm,
               src_v, dst_v, db_buf, eh_buf, ce_buf, eij_buf, sig_buf,
               nv_buf, stat_buf, zbuf, num_acc, den_acc, sem1, sem2):
    f32 = jnp.float32
    zero16 = jnp.zeros((16,), f32)

    def zrow(r, _):
        for k in range(4):
            zbuf[r, pl.ds(k * 16, 16)] = zero16
        return 0

    lax.fori_loop(0, ZROWS, zrow, 0)
    for t in range(N_PER_SUB // ZROWS):
        row0 = pl.multiple_of(sub * N_PER_SUB + t * ZROWS, 8)
        pltpu.sync_copy(zbuf, num_acc.at[pl.ds(row0, ZROWS)])
        pltpu.sync_copy(zbuf, den_acc.at[pl.ds(row0, ZROWS)])
    plsc.subcore_barrier()

    def block(i, carry):
        base = pl.multiple_of(sub * E_PER_SUB + i * EB, 8)
        pltpu.sync_copy(src_hbm.at[pl.ds(base, EB)], src_v)
        pltpu.sync_copy(dst_hbm.at[pl.ds(base, EB)], dst_v)
        g1 = pltpu.async_copy(db_tab.at[src_v], db_buf, sem1)
        g2 = pltpu.async_copy(eh_tab.at[dst_v], eh_buf, sem2)
        pltpu.sync_copy(ce_hbm.at[pl.ds(base, EB)], ce_buf)
        g1.wait()
        g2.wait()

        def row(r, acc):
            s0, s1, s2, s3, q0, q1, q2, q3 = acc
            ss = [s0, s1, s2, s3]
            qq = [q0, q1, q2, q3]
            for k in range(4):
                sl = pl.ds(k * 16, 16)
                eij = ce_buf[r, sl] + db_buf[r, sl] + eh_buf[r, sl]
                eij_buf[r, sl] = eij
                sig = 1.0 / (1.0 + jnp.exp(-eij))
                sig_buf[r, sl] = sig
                nv_buf[r, sl] = sig * db_buf[r, pl.ds(H + k * 16, 16)]
                ss[k] = ss[k] + eij
                qq[k] = qq[k] + eij * eij
            return (ss[0], ss[1], ss[2], ss[3], qq[0], qq[1], qq[2], qq[3])

        acc = lax.fori_loop(0, EB, row, carry)
        pltpu.sync_copy(eij_buf, eij_hbm.at[pl.ds(base, EB)])
        pltpu.sync_copy(nv_buf, num_acc.at[dst_v], add=True)
        pltpu.sync_copy(sig_buf, den_acc.at[dst_v], add=True)
        return acc

    acc0 = (zero16,) * 8
    acc = lax.fori_loop(0, NBLK, block, acc0)
    for k in range(4):
        stat_buf[pl.ds(k * 16, 16)] = acc[k]
        stat_buf[pl.ds(H + k * 16, 16)] = acc[4 + k]
    pltpu.sync_copy(stat_buf, est_hbm.at[sub])

    plsc.subcore_barrier()
    rows = pl.multiple_of(sub * N_PER_SUB, 8)
    pltpu.sync_copy(num_acc.at[pl.ds(rows, N_PER_SUB)],
                    num_hbm.at[pl.ds(rows, N_PER_SUB)])
    pltpu.sync_copy(den_acc.at[pl.ds(rows, N_PER_SUB)],
                    den_hbm.at[pl.ds(rows, N_PER_SUB)])


def _edge_body(src_hbm, dst_hbm, db0, db1, ehtab0, ehtab1, ce0, ce1,
               eij0, eij1, num0, num1, den0, den1, est0, est1,
               src_v, dst_v, db_buf, eh_buf, ce_buf, eij_buf, sig_buf,
               nv_buf, stat_buf, zbuf, num_acc, den_acc, sem1, sem2):
    c = lax.axis_index("c")
    s = lax.axis_index("s")
    scratch = (src_v, dst_v, db_buf, eh_buf, ce_buf, eij_buf, sig_buf,
               nv_buf, stat_buf, zbuf, num_acc, den_acc, sem1, sem2)

    @pl.when(c == 0)
    def _():
        _edge_half(s, src_hbm, dst_hbm, db0, ehtab0, ce0, eij0, num0, den0,
                   est0, *scratch)

    @pl.when(c == 1)
    def _():
        _edge_half(s, src_hbm, dst_hbm, db1, ehtab1, ce1, eij1, num1, den1,
                   est1, *scratch)


def _edge_pass(src, dst, DB0, DB1, EhT0, EhT1, Ce0, Ce1):
    f32 = jnp.float32
    i32 = jnp.int32
    mesh = plsc.VectorSubcoreMesh(core_axis_name="c", subcore_axis_name="s")
    kern = pl.kernel(
        _edge_body,
        out_type=[
            jax.ShapeDtypeStruct((E, H), f32),
            jax.ShapeDtypeStruct((E, H), f32),
            jax.ShapeDtypeStruct((N, H), f32),
            jax.ShapeDtypeStruct((N, H), f32),
            jax.ShapeDtypeStruct((N, H), f32),
            jax.ShapeDtypeStruct((N, H), f32),
            jax.ShapeDtypeStruct((NSUB, D), f32),
            jax.ShapeDtypeStruct((NSUB, D), f32),
        ],
        mesh=mesh,
        scratch_types=[
            pltpu.VMEM((EB,), i32),
            pltpu.VMEM((EB,), i32),
            pltpu.VMEM((EB, D), f32),
            pltpu.VMEM((EB, H), f32),
            pltpu.VMEM((EB, H), f32),
            pltpu.VMEM((EB, H), f32),
            pltpu.VMEM((EB, H), f32),
            pltpu.VMEM((EB, H), f32),
            pltpu.VMEM((D,), f32),
            pltpu.VMEM((ZROWS, H), f32),
            pltpu.VMEM_SHARED((N, H), f32),
            pltpu.VMEM_SHARED((N, H), f32),
            pltpu.SemaphoreType.DMA,
            pltpu.SemaphoreType.DMA,
        ],
    )
    return kern(src, dst, DB0, DB1, EhT0, EhT1, Ce0, Ce1)


def _hpath_body(h_ref, ah_ref, num0_ref, num1_ref, den0_ref, den1_ref,
                gh_ref, bh_ref, est0_ref, est1_ref, ge_ref, be_ref,
                hout_ref, ebn_ref):
    inv_e = 1.0 / E
    num = jnp.concatenate([num0_ref[...], num1_ref[...]], axis=1)
    den = jnp.concatenate([den0_ref[...], den1_ref[...]], axis=1)
    h_new = ah_ref[...] + num / (den + EPS_DEN)
    mu = jnp.mean(h_new, axis=0, keepdims=True)
    var = jnp.mean(h_new * h_new, axis=0, keepdims=True) - mu * mu
    y = (h_new - mu) * jax.lax.rsqrt(var + EPS_BN) * gh_ref[...] + bh_ref[...]
    hout_ref[...] = h_ref[...] + jnp.maximum(y, 0.0)

    st0 = jnp.sum(est0_ref[...], axis=0, keepdims=True)
    st1 = jnp.sum(est1_ref[...], axis=0, keepdims=True)
    mu0 = st0[:, :H] * inv_e
    mu1 = st1[:, :H] * inv_e
    var0 = st0[:, H:] * inv_e - mu0 * mu0
    var1 = st1[:, H:] * inv_e - mu1 * mu1
    sc0 = ge_ref[:, :H] * jax.lax.rsqrt(var0 + EPS_BN)
    sc1 = ge_ref[:, H:] * jax.lax.rsqrt(var1 + EPS_BN)
    sh0 = be_ref[:, :H] - mu0 * sc0
    sh1 = be_ref[:, H:] - mu1 * sc1
    ebn_ref[...] = jnp.concatenate([sc0, sc1, sh0, sh1], axis=0)


def _hpath(h, Ah, num0, num1, den0, den1, gamma_h, beta_h, est0, est1,
           gamma_e, beta_e):
    f32 = jnp.float32
    return pl.pallas_call(
        _hpath_body,
        out_shape=[
            jax.ShapeDtypeStruct((N, D), f32),
            jax.ShapeDtypeStruct((4, H), f32),
        ],
    )(h, Ah, num0, num1, den0, den1, gamma_h.reshape(1, D),
      beta_h.reshape(1, D), est0, est1, gamma_e.reshape(1, D),
      beta_e.reshape(1, D))


EO_BLK = 2000


def _epath_body(e_ref, eij0_ref, eij1_ref, ebn_ref, eout_ref):
    sc0 = ebn_ref[0:1, :]
    sc1 = ebn_ref[1:2, :]
    sh0 = ebn_ref[2:3, :]
    sh1 = ebn_ref[3:4, :]
    y0 = jnp.maximum(eij0_ref[...] * sc0 + sh0, 0.0)
    y1 = jnp.maximum(eij1_ref[...] * sc1 + sh1, 0.0)
    eout_ref[...] = e_ref[...] + jnp.concatenate([y0, y1], axis=1)


def _epath(e, Eij0, Eij1, ebn):
    f32 = jnp.float32
    grid = E // EO_BLK
    return pl.pallas_call(
        _epath_body,
        grid=(grid,),
        in_specs=[
            pl.BlockSpec((EO_BLK, D), lambda i: (i, 0)),
            pl.BlockSpec((EO_BLK, H), lambda i: (i, 0)),
            pl.BlockSpec((EO_BLK, H), lambda i: (i, 0)),
            pl.BlockSpec((4, H), lambda i: (0, 0)),
        ],
        out_specs=pl.BlockSpec((EO_BLK, D), lambda i: (i, 0)),
        out_shape=jax.ShapeDtypeStruct((E, D), f32),
    )(e, Eij0, Eij1, ebn)


def kernel(h, e, edge_index, W_A, b_A, W_B, b_B, W_C, b_C, W_D, b_D, W_E,
           b_E, gamma_h, beta_h, gamma_e, beta_e):
    src = edge_index[0]
    dst = edge_index[1]

    W_DB0 = jnp.concatenate([W_D[:, :H], W_B[:, :H]], axis=1)
    b_DB0 = jnp.concatenate([b_D[:H], b_B[:H]])
    W_DB1 = jnp.concatenate([W_D[:, H:], W_B[:, H:]], axis=1)
    b_DB1 = jnp.concatenate([b_D[H:], b_B[H:]])

    Ah, DB0, DB1, EhT0, EhT1 = _node_linears(
        h, W_A, b_A, W_DB0, b_DB0, W_DB1, b_DB1, W_E[:, :H], b_E[:H],
        W_E[:, H:], b_E[H:])
    Ce0, Ce1 = _ce_linears(e, W_C[:, :H], b_C[:H], W_C[:, H:], b_C[H:])

    Eij0, Eij1, num0, num1, den0, den1, est0, est1 = _edge_pass(
        src, dst, DB0, DB1, EhT0, EhT1, Ce0, Ce1)

    h_out, ebn = _hpath(h, Ah, num0, num1, den0, den1, gamma_h, beta_h,
                        est0, est1, gamma_e, beta_e)
    e_out = _epath(e, Eij0, Eij1, ebn)
    return (h_out, e_out)


# speedup vs baseline: 1.4452x; 1.4452x over previous
import jax
import jax.numpy as jnp
from jax import lax
from jax.experimental import pallas as pl
from jax.experimental.pallas import tpu as pltpu
from jax.experimental.pallas import tpu_sc as plsc

N = 10000
E = 320000
D = 128
NSUB = 16
EB = 80
EPS_DEN = 1e-6
EPS_BN = 1e-5

NW = 2 * NSUB
E_PER_W = E // NW
NBLK_W = E_PER_W // EB
N_PAD = 10240
ZROWS = 40


def _node_linear_body(h_ref, wa_ref, ba_ref, wb_ref, bb_ref, wd_ref,
                      bd_ref, we_ref, be_ref,
                      ah_ref, eh_ref, dbh_ref):
    hv = h_ref[...]
    f32 = jnp.float32
    ah_ref[...] = jnp.dot(hv, wa_ref[...],
                          preferred_element_type=f32) + ba_ref[...]
    eh_ref[...] = jnp.dot(hv, we_ref[...],
                          preferred_element_type=f32) + be_ref[...]
    dh = jnp.dot(hv, wd_ref[...], preferred_element_type=f32) + bd_ref[...]
    bh = jnp.dot(hv, wb_ref[...], preferred_element_type=f32) + bb_ref[...]
    dbh_ref[...] = jnp.concatenate([dh, bh], axis=1)


def _node_linears(h, W_A, b_A, W_B, b_B, W_D, b_D, W_E, b_E):
    f32 = jnp.float32
    return pl.pallas_call(
        _node_linear_body,
        out_shape=[
            jax.ShapeDtypeStruct((N, D), f32),
            jax.ShapeDtypeStruct((N, D), f32),
            jax.ShapeDtypeStruct((N, 2 * D), f32),
        ],
    )(h, W_A, b_A.reshape(1, D), W_B, b_B.reshape(1, D), W_D,
      b_D.reshape(1, D), W_E, b_E.reshape(1, D))


CE_BLK = 2000


def _ce_body(e_ref, wc_ref, bc_ref, ce_ref):
    ce_ref[...] = jnp.dot(e_ref[...], wc_ref[...],
                          preferred_element_type=jnp.float32) + bc_ref[...]


def _ce_linear(e, W_C, b_C):
    f32 = jnp.float32
    grid = E // CE_BLK
    return pl.pallas_call(
        _ce_body,
        grid=(grid,),
        in_specs=[
            pl.BlockSpec((CE_BLK, D), lambda i: (i, 0)),
            pl.BlockSpec((D, D), lambda i: (0, 0)),
            pl.BlockSpec((1, D), lambda i: (0, 0)),
        ],
        out_specs=pl.BlockSpec((CE_BLK, D), lambda i: (i, 0)),
        out_shape=jax.ShapeDtypeStruct((E, D), f32),
    )(e, W_C, b_C.reshape(1, D))


def _zero_acc(sub, zbuf, acc):
    zero16 = jnp.zeros((16,), jnp.float32)

    def zrow(r, _):
        for k in range(D // 16):
            zbuf[r, pl.ds(k * 16, 16)] = zero16
        return 0

    lax.fori_loop(0, ZROWS, zrow, 0)
    for t in range(N_PAD // NSUB // ZROWS):
        row0 = pl.multiple_of(sub * (N_PAD // NSUB) + t * ZROWS, 8)
        pltpu.sync_copy(zbuf, acc.at[pl.ds(row0, ZROWS)])
    plsc.subcore_barrier()


def _publish_acc(core, sub, acc, out_hbm):
    plsc.subcore_barrier()
    rows = pl.multiple_of(sub * (N_PAD // NSUB), 8)
    pltpu.sync_copy(
        acc.at[pl.ds(rows, N_PAD // NSUB)],
        out_hbm.at[pl.ds(pl.multiple_of(core * N_PAD, 8) + rows,
                         N_PAD // NSUB)])


def _edge_num_body(src_hbm, dst_hbm, dbh_tab, eh_tab, ce_hbm,
                   eij_hbm, num_hbm, est_hbm,
                   src_v, dst_v, dbh_buf, eh_buf, ce_buf,
                   stat_buf, zbuf, acc, sem1, sem2):
    c = lax.axis_index("c")
    s = lax.axis_index("s")
    w = c * NSUB + s
    zero16 = jnp.zeros((16,), jnp.float32)
    _zero_acc(s, zbuf, acc)

    def block(i, carry):
        base = pl.multiple_of(w * E_PER_W + i * EB, 8)
        pltpu.sync_copy(src_hbm.at[pl.ds(base, EB)], src_v)
        pltpu.sync_copy(dst_hbm.at[pl.ds(base, EB)], dst_v)
        g1 = pltpu.async_copy(dbh_tab.at[src_v], dbh_buf, sem1)
        g2 = pltpu.async_copy(eh_tab.at[dst_v], eh_buf, sem2)
        pltpu.sync_copy(ce_hbm.at[pl.ds(base, EB)], ce_buf)
        g1.wait()
        g2.wait()

        def row(r, acc8):
            out = list(acc8)
            for k in range(D // 16):
                sl = pl.ds(k * 16, 16)
                eij = ce_buf[r, sl] + dbh_buf[r, sl] + eh_buf[r, sl]
                ce_buf[r, sl] = eij
                sig = 1.0 / (1.0 + jnp.exp(-eij))
                eh_buf[r, sl] = sig * dbh_buf[r, pl.ds(D + k * 16, 16)]
                out[k] = out[k] + eij
                out[8 + k] = out[8 + k] + eij * eij
            return tuple(out)

        acc8 = lax.fori_loop(0, EB, row, carry)
        pltpu.sync_copy(ce_buf, eij_hbm.at[pl.ds(base, EB)])
        pltpu.sync_copy(eh_buf, acc.at[dst_v], add=True)
        return acc8

    acc8 = lax.fori_loop(0, NBLK_W, block, (zero16,) * 16)
    for r in range(8):
        for k in range(D // 16):
            stat_buf[r, pl.ds(k * 16, 16)] = zero16
    for k in range(8):
        stat_buf[0, pl.ds(k * 16, 16)] = acc8[k]
        stat_buf[1, pl.ds(k * 16, 16)] = acc8[8 + k]
    pltpu.sync_copy(stat_buf, est_hbm.at[pl.ds(pl.multiple_of(w * 8, 8), 8)])
    _publish_acc(c, s, acc, num_hbm)


def _edge_num(src, dst, DBh, Eh, Ce):
    f32 = jnp.float32
    i32 = jnp.int32
    mesh = plsc.VectorSubcoreMesh(core_axis_name="c", subcore_axis_name="s")
    kern = pl.kernel(
        _edge_num_body,
        out_type=[
            jax.ShapeDtypeStruct((E, D), f32),
            jax.ShapeDtypeStruct((2 * N_PAD, D), f32),
            jax.ShapeDtypeStruct((NW * 8, D), f32),
        ],
        mesh=mesh,
        scratch_types=[
            pltpu.VMEM((EB,), i32),
            pltpu.VMEM((EB,), i32),
            pltpu.VMEM((EB, 2 * D), f32),
            pltpu.VMEM((EB, D), f32),
            pltpu.VMEM((EB, D), f32),
            pltpu.VMEM((8, D), f32),
            pltpu.VMEM((ZROWS, D), f32),
            pltpu.VMEM_SHARED((N_PAD, D), f32),
            pltpu.SemaphoreType.DMA,
            pltpu.SemaphoreType.DMA,
        ],
    )
    return kern(src, dst, DBh, Eh, Ce)


def _den_body(dst_hbm, eij_hbm, den_hbm,
              dst_v, eij_buf, zbuf, acc):
    c = lax.axis_index("c")
    s = lax.axis_index("s")
    w = c * NSUB + s
    _zero_acc(s, zbuf, acc)

    def block(i, _):
        base = pl.multiple_of(w * E_PER_W + i * EB, 8)
        pltpu.sync_copy(dst_hbm.at[pl.ds(base, EB)], dst_v)
        pltpu.sync_copy(eij_hbm.at[pl.ds(base, EB)], eij_buf)

        def row(r, carry):
            for k in range(D // 16):
                sl = pl.ds(k * 16, 16)
                eij_buf[r, sl] = 1.0 / (1.0 + jnp.exp(-eij_buf[r, sl]))
            return carry

        lax.fori_loop(0, EB, row, 0)
        pltpu.sync_copy(eij_buf, acc.at[dst_v], add=True)
        return 0

    lax.fori_loop(0, NBLK_W, block, 0)
    _publish_acc(c, s, acc, den_hbm)


def _den_scatter(dst, Eij):
    f32 = jnp.float32
    i32 = jnp.int32
    mesh = plsc.VectorSubcoreMesh(core_axis_name="c", subcore_axis_name="s")
    kern = pl.kernel(
        _den_body,
        out_type=jax.ShapeDtypeStruct((2 * N_PAD, D), f32),
        mesh=mesh,
        scratch_types=[
            pltpu.VMEM((EB,), i32),
            pltpu.VMEM((EB, D), f32),
            pltpu.VMEM((ZROWS, D), f32),
            pltpu.VMEM_SHARED((N_PAD, D), f32),
        ],
    )
    return kern(dst, Eij)


def _hpath_body(h_ref, ah_ref, num_ref, den_ref, gh_ref, bh_ref,
                est_ref, ge_ref, be_ref, hout_ref, ebn_ref):
    inv_e = 1.0 / E
    num = num_ref[:N, :] + num_ref[N_PAD:N_PAD + N, :]
    den = den_ref[:N, :] + den_ref[N_PAD:N_PAD + N, :]
    h_new = ah_ref[...] + num / (den + EPS_DEN)
    mu = jnp.mean(h_new, axis=0, keepdims=True)
    var = jnp.mean(h_new * h_new, axis=0, keepdims=True) - mu * mu
    y = (h_new - mu) * jax.lax.rsqrt(var + EPS_BN) * gh_ref[...] + bh_ref[...]
    hout_ref[...] = h_ref[...] + jnp.maximum(y, 0.0)

    est = est_ref[...]
    rows = jax.lax.broadcasted_iota(jnp.int32, est.shape, 0)
    s_e = jnp.sum(jnp.where(rows % 8 == 0, est, 0.0), axis=0, keepdims=True)
    q_e = jnp.sum(jnp.where(rows % 8 == 1, est, 0.0), axis=0, keepdims=True)
    mu_e = s_e * inv_e
    var_e = q_e * inv_e - mu_e * mu_e
    sc_e = ge_ref[...] * jax.lax.rsqrt(var_e + EPS_BN)
    sh_e = be_ref[...] - mu_e * sc_e
    pad = jnp.zeros((6, D), jnp.float32)
    ebn_ref[...] = jnp.concatenate([sc_e, sh_e, pad], axis=0)


def _hpath(h, Ah, num, den, gamma_h, beta_h, est, gamma_e, beta_e):
    f32 = jnp.float32
    return pl.pallas_call(
        _hpath_body,
        out_shape=[
            jax.ShapeDtypeStruct((N, D), f32),
            jax.ShapeDtypeStruct((8, D), f32),
        ],
    )(h, Ah, num, den, gamma_h.reshape(1, D), beta_h.reshape(1, D),
      est, gamma_e.reshape(1, D), beta_e.reshape(1, D))


EO_BLK = 2000


def _epath_body(e_ref, eij_ref, ebn_ref, eout_ref):
    sc_e = ebn_ref[0:1, :]
    sh_e = ebn_ref[1:2, :]
    y = jnp.maximum(eij_ref[...] * sc_e + sh_e, 0.0)
    eout_ref[...] = e_ref[...] + y


def _epath(e, Eij, ebn):
    f32 = jnp.float32
    grid = E // EO_BLK
    return pl.pallas_call(
        _epath_body,
        grid=(grid,),
        in_specs=[
            pl.BlockSpec((EO_BLK, D), lambda i: (i, 0)),
            pl.BlockSpec((EO_BLK, D), lambda i: (i, 0)),
            pl.BlockSpec((8, D), lambda i: (0, 0)),
        ],
        out_specs=pl.BlockSpec((EO_BLK, D), lambda i: (i, 0)),
        out_shape=jax.ShapeDtypeStruct((E, D), f32),
    )(e, Eij, ebn)


def kernel(h, e, edge_index, W_A, b_A, W_B, b_B, W_C, b_C, W_D, b_D, W_E,
           b_E, gamma_h, beta_h, gamma_e, beta_e):
    src = edge_index[0]
    dst = edge_index[1]

    Ah, Eh, DBh = _node_linears(h, W_A, b_A, W_B, b_B, W_D, b_D, W_E, b_E)
    Ce = _ce_linear(e, W_C, b_C)

    Eij, num, est = _edge_num(src, dst, DBh, Eh, Ce)
    den = _den_scatter(dst, Eij)

    h_out, ebn = _hpath(h, Ah, num, den, gamma_h, beta_h, est,
                        gamma_e, beta_e)
    e_out = _epath(e, Eij, ebn)
    return (h_out, e_out)


# speedup vs baseline: 2.8870x; 1.9977x over previous
import jax
import jax.numpy as jnp
from jax import lax
from jax.experimental import pallas as pl
from jax.experimental.pallas import tpu as pltpu
from jax.experimental.pallas import tpu_sc as plsc

N = 10000
E = 320000
D = 128
NSUB = 16
EB = 80
EPS_DEN = 1e-6
EPS_BN = 1e-5

NW = 2 * NSUB
E_PER_W = E // NW
NBLK_W = E_PER_W // EB
N_PAD = 10240
ZROWS = 40


def _node_linear_body(h_ref, wa_ref, ba_ref, wb_ref, bb_ref, wd_ref,
                      bd_ref, we_ref, be_ref,
                      ah_ref, eh_ref, dh_ref, bh_ref):
    hv = h_ref[...]
    f32 = jnp.float32
    ah_ref[...] = jnp.dot(hv, wa_ref[...],
                          preferred_element_type=f32) + ba_ref[...]
    eh_ref[...] = jnp.dot(hv, we_ref[...],
                          preferred_element_type=f32) + be_ref[...]
    dh_ref[...] = jnp.dot(hv, wd_ref[...],
                          preferred_element_type=f32) + bd_ref[...]
    bh_ref[...] = jnp.dot(hv, wb_ref[...],
                          preferred_element_type=f32) + bb_ref[...]


def _node_linears(h, W_A, b_A, W_B, b_B, W_D, b_D, W_E, b_E):
    f32 = jnp.float32
    return pl.pallas_call(
        _node_linear_body,
        out_shape=[
            jax.ShapeDtypeStruct((N, D), f32),
            jax.ShapeDtypeStruct((N, D), f32),
            jax.ShapeDtypeStruct((N, D), f32),
            jax.ShapeDtypeStruct((N, D), f32),
        ],
    )(h, W_A, b_A.reshape(1, D), W_B, b_B.reshape(1, D), W_D,
      b_D.reshape(1, D), W_E, b_E.reshape(1, D))


CE_BLK = 2000


def _ce_body(e_ref, wc_ref, bc_ref, ce_ref):
    ce_ref[...] = jnp.dot(e_ref[...], wc_ref[...],
                          preferred_element_type=jnp.float32) + bc_ref[...]


def _ce_linear(e, W_C, b_C):
    f32 = jnp.float32
    grid = E // CE_BLK
    return pl.pallas_call(
        _ce_body,
        grid=(grid,),
        in_specs=[
            pl.BlockSpec((CE_BLK, D), lambda i: (i, 0)),
            pl.BlockSpec((D, D), lambda i: (0, 0)),
            pl.BlockSpec((1, D), lambda i: (0, 0)),
        ],
        out_specs=pl.BlockSpec((CE_BLK, D), lambda i: (i, 0)),
        out_shape=jax.ShapeDtypeStruct((E, D), f32),
    )(e, W_C, b_C.reshape(1, D))


def _zero_acc(sub, zbuf, acc):
    zero16 = jnp.zeros((16,), jnp.float32)

    def zrow(r, _):
        for k in range(D // 16):
            zbuf[r, pl.ds(k * 16, 16)] = zero16
        return 0

    lax.fori_loop(0, ZROWS, zrow, 0)
    for t in range(N_PAD // NSUB // ZROWS):
        row0 = pl.multiple_of(sub * (N_PAD // NSUB) + t * ZROWS, 8)
        pltpu.sync_copy(zbuf, acc.at[pl.ds(row0, ZROWS)])
    plsc.subcore_barrier()


def _publish_acc(core, sub, acc, out_hbm):
    plsc.subcore_barrier()
    rows = pl.multiple_of(sub * (N_PAD // NSUB), 8)
    pltpu.sync_copy(
        acc.at[pl.ds(rows, N_PAD // NSUB)],
        out_hbm.at[pl.ds(pl.multiple_of(core * N_PAD, 8) + rows,
                         N_PAD // NSUB)])


def _edge_num_body(src_hbm, dst_hbm, dh_tab, bh_tab, eh_tab, ce_hbm,
                   eij_hbm, num_hbm, est_hbm,
                   src_v, dst_v, dh_buf, bh_buf, eh_buf, ce_buf,
                   stat_buf, zbuf, acc, sem1, sem2, sem3):
    c = lax.axis_index("c")
    s = lax.axis_index("s")
    w = c * NSUB + s
    zero16 = jnp.zeros((16,), jnp.float32)
    _zero_acc(s, zbuf, acc)

    def block(i, carry):
        base = pl.multiple_of(w * E_PER_W + i * EB, 8)
        pltpu.sync_copy(src_hbm.at[pl.ds(base, EB)], src_v)
        pltpu.sync_copy(dst_hbm.at[pl.ds(base, EB)], dst_v)
        g1 = pltpu.async_copy(dh_tab.at[src_v], dh_buf, sem1)
        g2 = pltpu.async_copy(eh_tab.at[dst_v], eh_buf, sem2)
        g3 = pltpu.async_copy(bh_tab.at[src_v], bh_buf, sem3)
        pltpu.sync_copy(ce_hbm.at[pl.ds(base, EB)], ce_buf)
        g1.wait()
        g2.wait()
        g3.wait()

        def row(r, acc8):
            out = list(acc8)
            for k in range(D // 16):
                sl = pl.ds(k * 16, 16)
                eij = ce_buf[r, sl] + dh_buf[r, sl] + eh_buf[r, sl]
                ce_buf[r, sl] = eij
                sig = 1.0 / (1.0 + jnp.exp(-eij))
                eh_buf[r, sl] = sig * bh_buf[r, sl]
                out[k] = out[k] + eij
                out[8 + k] = out[8 + k] + eij * eij
            return tuple(out)

        acc8 = lax.fori_loop(0, EB, row, carry)
        pltpu.sync_copy(ce_buf, eij_hbm.at[pl.ds(base, EB)])
        pltpu.sync_copy(eh_buf, acc.at[dst_v], add=True)
        return acc8

    acc8 = lax.fori_loop(0, NBLK_W, block, (zero16,) * 16)
    for r in range(8):
        for k in range(D // 16):
            stat_buf[r, pl.ds(k * 16, 16)] = zero16
    for k in range(8):
        stat_buf[0, pl.ds(k * 16, 16)] = acc8[k]
        stat_buf[1, pl.ds(k * 16, 16)] = acc8[8 + k]
    pltpu.sync_copy(stat_buf, est_hbm.at[pl.ds(pl.multiple_of(w * 8, 8), 8)])
    _publish_acc(c, s, acc, num_hbm)


def _edge_num(src, dst, Dh, Bh, Eh, Ce):
    f32 = jnp.float32
    i32 = jnp.int32
    mesh = plsc.VectorSubcoreMesh(core_axis_name="c", subcore_axis_name="s")
    kern = pl.kernel(
        _edge_num_body,
        out_type=[
            jax.ShapeDtypeStruct((E, D), f32),
            jax.ShapeDtypeStruct((2 * N_PAD, D), f32),
            jax.ShapeDtypeStruct((NW * 8, D), f32),
        ],
        mesh=mesh,
        scratch_types=[
            pltpu.VMEM((EB,), i32),
            pltpu.VMEM((EB,), i32),
            pltpu.VMEM((EB, D), f32),
            pltpu.VMEM((EB, D), f32),
            pltpu.VMEM((EB, D), f32),
            pltpu.VMEM((EB, D), f32),
            pltpu.VMEM((8, D), f32),
            pltpu.VMEM((ZROWS, D), f32),
            pltpu.VMEM_SHARED((N_PAD, D), f32),
            pltpu.SemaphoreType.DMA,
            pltpu.SemaphoreType.DMA,
            pltpu.SemaphoreType.DMA,
        ],
    )
    return kern(src, dst, Dh, Bh, Eh, Ce)


def _den_body(dst_hbm, eij_hbm, den_hbm,
              dst_v, eij_buf, zbuf, acc):
    c = lax.axis_index("c")
    s = lax.axis_index("s")
    w = c * NSUB + s
    _zero_acc(s, zbuf, acc)

    def block(i, _):
        base = pl.multiple_of(w * E_PER_W + i * EB, 8)
        pltpu.sync_copy(dst_hbm.at[pl.ds(base, EB)], dst_v)
        pltpu.sync_copy(eij_hbm.at[pl.ds(base, EB)], eij_buf)

        def row(r, carry):
            for k in range(D // 16):
                sl = pl.ds(k * 16, 16)
                eij_buf[r, sl] = 1.0 / (1.0 + jnp.exp(-eij_buf[r, sl]))
            return carry

        lax.fori_loop(0, EB, row, 0)
        pltpu.sync_copy(eij_buf, acc.at[dst_v], add=True)
        return 0

    lax.fori_loop(0, NBLK_W, block, 0)
    _publish_acc(c, s, acc, den_hbm)


def _den_scatter(dst, Eij):
    f32 = jnp.float32
    i32 = jnp.int32
    mesh = plsc.VectorSubcoreMesh(core_axis_name="c", subcore_axis_name="s")
    kern = pl.kernel(
        _den_body,
        out_type=jax.ShapeDtypeStruct((2 * N_PAD, D), f32),
        mesh=mesh,
        scratch_types=[
            pltpu.VMEM((EB,), i32),
            pltpu.VMEM((EB, D), f32),
            pltpu.VMEM((ZROWS, D), f32),
            pltpu.VMEM_SHARED((N_PAD, D), f32),
        ],
    )
    return kern(dst, Eij)


def _hpath_body(h_ref, ah_ref, num_ref, den_ref, gh_ref, bh_ref,
                est_ref, ge_ref, be_ref, hout_ref, ebn_ref):
    inv_e = 1.0 / E
    num = num_ref[:N, :] + num_ref[N_PAD:N_PAD + N, :]
    den = den_ref[:N, :] + den_ref[N_PAD:N_PAD + N, :]
    h_new = ah_ref[...] + num / (den + EPS_DEN)
    mu = jnp.mean(h_new, axis=0, keepdims=True)
    var = jnp.mean(h_new * h_new, axis=0, keepdims=True) - mu * mu
    y = (h_new - mu) * jax.lax.rsqrt(var + EPS_BN) * gh_ref[...] + bh_ref[...]
    hout_ref[...] = h_ref[...] + jnp.maximum(y, 0.0)

    est = est_ref[...]
    rows = jax.lax.broadcasted_iota(jnp.int32, est.shape, 0)
    s_e = jnp.sum(jnp.where(rows % 8 == 0, est, 0.0), axis=0, keepdims=True)
    q_e = jnp.sum(jnp.where(rows % 8 == 1, est, 0.0), axis=0, keepdims=True)
    mu_e = s_e * inv_e
    var_e = q_e * inv_e - mu_e * mu_e
    sc_e = ge_ref[...] * jax.lax.rsqrt(var_e + EPS_BN)
    sh_e = be_ref[...] - mu_e * sc_e
    pad = jnp.zeros((6, D), jnp.float32)
    ebn_ref[...] = jnp.concatenate([sc_e, sh_e, pad], axis=0)


def _hpath(h, Ah, num, den, gamma_h, beta_h, est, gamma_e, beta_e):
    f32 = jnp.float32
    return pl.pallas_call(
        _hpath_body,
        out_shape=[
            jax.ShapeDtypeStruct((N, D), f32),
            jax.ShapeDtypeStruct((8, D), f32),
        ],
    )(h, Ah, num, den, gamma_h.reshape(1, D), beta_h.reshape(1, D),
      est, gamma_e.reshape(1, D), beta_e.reshape(1, D))


EO_BLK = 2000


def _epath_body(e_ref, eij_ref, ebn_ref, eout_ref):
    sc_e = ebn_ref[0:1, :]
    sh_e = ebn_ref[1:2, :]
    y = jnp.maximum(eij_ref[...] * sc_e + sh_e, 0.0)
    eout_ref[...] = e_ref[...] + y


def _epath(e, Eij, ebn):
    f32 = jnp.float32
    grid = E // EO_BLK
    return pl.pallas_call(
        _epath_body,
        grid=(grid,),
        in_specs=[
            pl.BlockSpec((EO_BLK, D), lambda i: (i, 0)),
            pl.BlockSpec((EO_BLK, D), lambda i: (i, 0)),
            pl.BlockSpec((8, D), lambda i: (0, 0)),
        ],
        out_specs=pl.BlockSpec((EO_BLK, D), lambda i: (i, 0)),
        out_shape=jax.ShapeDtypeStruct((E, D), f32),
    )(e, Eij, ebn)


def kernel(h, e, edge_index, W_A, b_A, W_B, b_B, W_C, b_C, W_D, b_D, W_E,
           b_E, gamma_h, beta_h, gamma_e, beta_e):
    src = edge_index[0]
    dst = edge_index[1]

    Ah, Eh, Dh, Bh = _node_linears(h, W_A, b_A, W_B, b_B, W_D, b_D, W_E, b_E)
    Ce = _ce_linear(e, W_C, b_C)

    Eij, num, est = _edge_num(src, dst, Dh, Bh, Eh, Ce)
    den = _den_scatter(dst, Eij)

    h_out, ebn = _hpath(h, Ah, num, den, gamma_h, beta_h, est,
                        gamma_e, beta_e)
    e_out = _epath(e, Eij, ebn)
    return (h_out, e_out)


# speedup vs baseline: 3.2008x; 1.1087x over previous
import jax
import jax.numpy as jnp
from jax import lax
from jax.experimental import pallas as pl
from jax.experimental.pallas import tpu as pltpu
from jax.experimental.pallas import tpu_sc as plsc

N = 10000
E = 320000
D = 128
NSUB = 16
EB = 80
EPS_DEN = 1e-6
EPS_BN = 1e-5

NW = 2 * NSUB
E_PER_W = E // NW
NBLK_W = E_PER_W // EB
N_PAD = 10240
ZROWS = 40


def _node_linear_body(h_ref, wa_ref, ba_ref, wb_ref, bb_ref, wd_ref,
                      bd_ref, we_ref, be_ref,
                      ah_ref, eh_ref, dh_ref, bh_ref):
    hv = h_ref[...]
    f32 = jnp.float32
    ah_ref[...] = jnp.dot(hv, wa_ref[...],
                          preferred_element_type=f32) + ba_ref[...]
    eh_ref[...] = jnp.dot(hv, we_ref[...],
                          preferred_element_type=f32) + be_ref[...]
    dh_ref[...] = jnp.dot(hv, wd_ref[...],
                          preferred_element_type=f32) + bd_ref[...]
    bh_ref[...] = jnp.dot(hv, wb_ref[...],
                          preferred_element_type=f32) + bb_ref[...]


def _node_linears(h, W_A, b_A, W_B, b_B, W_D, b_D, W_E, b_E):
    f32 = jnp.float32
    return pl.pallas_call(
        _node_linear_body,
        out_shape=[
            jax.ShapeDtypeStruct((N, D), f32),
            jax.ShapeDtypeStruct((N, D), f32),
            jax.ShapeDtypeStruct((N, D), f32),
            jax.ShapeDtypeStruct((N, D), f32),
        ],
    )(h, W_A, b_A.reshape(1, D), W_B, b_B.reshape(1, D), W_D,
      b_D.reshape(1, D), W_E, b_E.reshape(1, D))


CE_BLK = 2000


def _ce_body(e_ref, wc_ref, bc_ref, ce_ref):
    ce_ref[...] = jnp.dot(e_ref[...], wc_ref[...],
                          preferred_element_type=jnp.float32) + bc_ref[...]


def _ce_linear(e, W_C, b_C):
    f32 = jnp.float32
    grid = E // CE_BLK
    return pl.pallas_call(
        _ce_body,
        grid=(grid,),
        in_specs=[
            pl.BlockSpec((CE_BLK, D), lambda i: (i, 0)),
            pl.BlockSpec((D, D), lambda i: (0, 0)),
            pl.BlockSpec((1, D), lambda i: (0, 0)),
        ],
        out_specs=pl.BlockSpec((CE_BLK, D), lambda i: (i, 0)),
        out_shape=jax.ShapeDtypeStruct((E, D), f32),
    )(e, W_C, b_C.reshape(1, D))


def _zero_acc(sub, zbuf, acc):
    zero16 = jnp.zeros((16,), jnp.float32)

    def zrow(r, _):
        for k in range(D // 16):
            zbuf[r, pl.ds(k * 16, 16)] = zero16
        return 0

    lax.fori_loop(0, ZROWS, zrow, 0)
    for t in range(N_PAD // NSUB // ZROWS):
        row0 = pl.multiple_of(sub * (N_PAD // NSUB) + t * ZROWS, 8)
        pltpu.sync_copy(zbuf, acc.at[pl.ds(row0, ZROWS)])
    plsc.subcore_barrier()


def _publish_acc(core, sub, acc, out_hbm):
    plsc.subcore_barrier()
    rows = pl.multiple_of(sub * (N_PAD // NSUB), 8)
    pltpu.sync_copy(
        acc.at[pl.ds(rows, N_PAD // NSUB)],
        out_hbm.at[pl.ds(pl.multiple_of(core * N_PAD, 8) + rows,
                         N_PAD // NSUB)])


def _edge_num_body(src_hbm, dst_hbm, dh_tab, bh_tab, eh_tab, ce_hbm,
                   eij_hbm, num_hbm, est_hbm,
                   src_v, dst_v, dh_buf, bh_buf, eh_buf, ce_buf,
                   stat_buf, zbuf, acc, sem1, sem2, sem3):
    c = lax.axis_index("c")
    s = lax.axis_index("s")
    w = c * NSUB + s
    zero16 = jnp.zeros((16,), jnp.float32)
    _zero_acc(s, zbuf, acc)

    def block(i, carry):
        base = pl.multiple_of(w * E_PER_W + i * EB, 8)
        pltpu.sync_copy(src_hbm.at[pl.ds(base, EB)], src_v)
        pltpu.sync_copy(dst_hbm.at[pl.ds(base, EB)], dst_v)
        g1 = pltpu.async_copy(dh_tab.at[src_v], dh_buf, sem1)
        g2 = pltpu.async_copy(eh_tab.at[dst_v], eh_buf, sem2)
        g3 = pltpu.async_copy(bh_tab.at[src_v], bh_buf, sem3)
        pltpu.sync_copy(ce_hbm.at[pl.ds(base, EB)], ce_buf)
        g1.wait()
        g2.wait()
        g3.wait()

        def row(r, acc8):
            out = list(acc8)
            for k in range(D // 16):
                sl = pl.ds(k * 16, 16)
                eij = ce_buf[r, sl] + dh_buf[r, sl] + eh_buf[r, sl]
                ce_buf[r, sl] = eij
                sig = 1.0 / (1.0 + jnp.exp(-eij))
                eh_buf[r, sl] = sig * bh_buf[r, sl]
                out[k] = out[k] + eij
                out[8 + k] = out[8 + k] + eij * eij
            return tuple(out)

        acc8 = lax.fori_loop(0, EB, row, carry)
        pltpu.sync_copy(ce_buf, eij_hbm.at[pl.ds(base, EB)])
        pltpu.sync_copy(eh_buf, acc.at[dst_v], add=True)
        return acc8

    acc8 = lax.fori_loop(0, NBLK_W, block, (zero16,) * 16)
    for r in range(8):
        for k in range(D // 16):
            stat_buf[r, pl.ds(k * 16, 16)] = zero16
    for k in range(8):
        stat_buf[0, pl.ds(k * 16, 16)] = acc8[k]
        stat_buf[1, pl.ds(k * 16, 16)] = acc8[8 + k]
    pltpu.sync_copy(stat_buf, est_hbm.at[pl.ds(pl.multiple_of(w * 8, 8), 8)])
    _publish_acc(c, s, acc, num_hbm)


def _edge_num(src, dst, Dh, Bh, Eh, Ce):
    f32 = jnp.float32
    i32 = jnp.int32
    mesh = plsc.VectorSubcoreMesh(core_axis_name="c", subcore_axis_name="s")
    kern = pl.kernel(
        _edge_num_body,
        out_type=[
            jax.ShapeDtypeStruct((E, D), f32),
            jax.ShapeDtypeStruct((2 * N_PAD, D), f32),
            jax.ShapeDtypeStruct((NW * 8, D), f32),
        ],
        mesh=mesh,
        scratch_types=[
            pltpu.VMEM((EB,), i32),
            pltpu.VMEM((EB,), i32),
            pltpu.VMEM((EB, D), f32),
            pltpu.VMEM((EB, D), f32),
            pltpu.VMEM((EB, D), f32),
            pltpu.VMEM((EB, D), f32),
            pltpu.VMEM((8, D), f32),
            pltpu.VMEM((ZROWS, D), f32),
            pltpu.VMEM_SHARED((N_PAD, D), f32),
            pltpu.SemaphoreType.DMA,
            pltpu.SemaphoreType.DMA,
            pltpu.SemaphoreType.DMA,
        ],
    )
    return kern(src, dst, Dh, Bh, Eh, Ce)


def _den_body(dst_hbm, eij_hbm, den_hbm,
              dst_v, eij_buf, zbuf, acc):
    c = lax.axis_index("c")
    s = lax.axis_index("s")
    w = c * NSUB + s
    _zero_acc(s, zbuf, acc)

    def block(i, _):
        base = pl.multiple_of(w * E_PER_W + i * EB, 8)
        pltpu.sync_copy(dst_hbm.at[pl.ds(base, EB)], dst_v)
        pltpu.sync_copy(eij_hbm.at[pl.ds(base, EB)], eij_buf)

        def row(r, carry):
            for k in range(D // 16):
                sl = pl.ds(k * 16, 16)
                eij_buf[r, sl] = 1.0 / (1.0 + jnp.exp(-eij_buf[r, sl]))
            return carry

        lax.fori_loop(0, EB, row, 0)
        pltpu.sync_copy(eij_buf, acc.at[dst_v], add=True)
        return 0

    lax.fori_loop(0, NBLK_W, block, 0)
    _publish_acc(c, s, acc, den_hbm)


def _den_scatter(dst, Eij):
    f32 = jnp.float32
    i32 = jnp.int32
    mesh = plsc.VectorSubcoreMesh(core_axis_name="c", subcore_axis_name="s")
    kern = pl.kernel(
        _den_body,
        out_type=jax.ShapeDtypeStruct((2 * N_PAD, D), f32),
        mesh=mesh,
        scratch_types=[
            pltpu.VMEM((EB,), i32),
            pltpu.VMEM((EB, D), f32),
            pltpu.VMEM((ZROWS, D), f32),
            pltpu.VMEM_SHARED((N_PAD, D), f32),
        ],
    )
    return kern(dst, Eij)


def _ebn_body(est_ref, ge_ref, be_ref, ebn_ref):
    inv_e = 1.0 / E
    est = est_ref[...]
    rows = jax.lax.broadcasted_iota(jnp.int32, est.shape, 0)
    s_e = jnp.sum(jnp.where(rows % 8 == 0, est, 0.0), axis=0, keepdims=True)
    q_e = jnp.sum(jnp.where(rows % 8 == 1, est, 0.0), axis=0, keepdims=True)
    mu_e = s_e * inv_e
    var_e = q_e * inv_e - mu_e * mu_e
    sc_e = ge_ref[...] * jax.lax.rsqrt(var_e + EPS_BN)
    sh_e = be_ref[...] - mu_e * sc_e
    pad = jnp.zeros((6, D), jnp.float32)
    ebn_ref[...] = jnp.concatenate([sc_e, sh_e, pad], axis=0)


def _ebn_reduce(est, gamma_e, beta_e):
    return pl.pallas_call(
        _ebn_body,
        out_shape=jax.ShapeDtypeStruct((8, D), jnp.float32),
    )(est, gamma_e.reshape(1, D), beta_e.reshape(1, D))


def _hpath_body(h_ref, ah_ref, num_ref, den_ref, gh_ref, bh_ref, hout_ref):
    num = num_ref[:N, :] + num_ref[N_PAD:N_PAD + N, :]
    den = den_ref[:N, :] + den_ref[N_PAD:N_PAD + N, :]
    h_new = ah_ref[...] + num / (den + EPS_DEN)
    mu = jnp.mean(h_new, axis=0, keepdims=True)
    var = jnp.mean(h_new * h_new, axis=0, keepdims=True) - mu * mu
    y = (h_new - mu) * jax.lax.rsqrt(var + EPS_BN) * gh_ref[...] + bh_ref[...]
    hout_ref[...] = h_ref[...] + jnp.maximum(y, 0.0)


def _hpath(h, Ah, num, den, gamma_h, beta_h):
    f32 = jnp.float32
    return pl.pallas_call(
        _hpath_body,
        out_shape=jax.ShapeDtypeStruct((N, D), f32),
    )(h, Ah, num, den, gamma_h.reshape(1, D), beta_h.reshape(1, D))


EO_BLK = 2000


def _epath_body(e_ref, eij_ref, ebn_ref, eout_ref):
    sc_e = ebn_ref[0:1, :]
    sh_e = ebn_ref[1:2, :]
    y = jnp.maximum(eij_ref[...] * sc_e + sh_e, 0.0)
    eout_ref[...] = e_ref[...] + y


def _epath(e, Eij, ebn):
    f32 = jnp.float32
    grid = E // EO_BLK
    return pl.pallas_call(
        _epath_body,
        grid=(grid,),
        in_specs=[
            pl.BlockSpec((EO_BLK, D), lambda i: (i, 0)),
            pl.BlockSpec((EO_BLK, D), lambda i: (i, 0)),
            pl.BlockSpec((8, D), lambda i: (0, 0)),
        ],
        out_specs=pl.BlockSpec((EO_BLK, D), lambda i: (i, 0)),
        out_shape=jax.ShapeDtypeStruct((E, D), f32),
    )(e, Eij, ebn)


def kernel(h, e, edge_index, W_A, b_A, W_B, b_B, W_C, b_C, W_D, b_D, W_E,
           b_E, gamma_h, beta_h, gamma_e, beta_e):
    src = edge_index[0]
    dst = edge_index[1]

    Ah, Eh, Dh, Bh = _node_linears(h, W_A, b_A, W_B, b_B, W_D, b_D, W_E, b_E)
    Ce = _ce_linear(e, W_C, b_C)

    Eij, num, est = _edge_num(src, dst, Dh, Bh, Eh, Ce)
    ebn = _ebn_reduce(est, gamma_e, beta_e)
    e_out = _epath(e, Eij, ebn)
    den = _den_scatter(dst, Eij)

    h_out = _hpath(h, Ah, num, den, gamma_h, beta_h)
    return (h_out, e_out)
